# baseline (speedup 1.0000x reference)
import functools

import jax
import jax.numpy as jnp
from jax import lax
from jax.experimental import pallas as pl
from jax.experimental.pallas import tpu as pltpu
from jax.experimental.pallas import tpu_sc as plsc

NC = 2
NS = 16
NW = NC * NS
CH = 640
DW = 16
NG0 = 16
NG1 = 16
NGT = NS * (NG0 + NG1)
NGMAX = max(NG0, NG1)


def _sc_mesh():
    return plsc.VectorSubcoreMesh(core_axis_name="c", subcore_axis_name="s")


@functools.lru_cache(maxsize=None)
def _make_deg(n_pad, gpw):
    zrows = n_pad // NS

    @functools.partial(
        pl.kernel,
        out_type=jax.ShapeDtypeStruct((NC * n_pad, DW), jnp.float32),
        mesh=_sc_mesh(),
        compiler_params=pltpu.CompilerParams(use_tc_tiling_on_sc=False),
        scratch_types=[
            pltpu.VMEM((gpw, CH), jnp.int32),
            pltpu.VMEM((CH, DW), jnp.float32),
            pltpu.VMEM_SHARED((n_pad, DW), jnp.float32),
            pltpu.SemaphoreType.DMA,
        ],
    )
    def deg_kernel(dst_hbm, ones_hbm, zero_hbm, out_hbm, didx, ones_v, acc, sem):
        cid = lax.axis_index("c")
        sid = lax.axis_index("s")
        wid = cid * NS + sid
        pltpu.sync_copy(zero_hbm.at[pl.ds(sid * zrows, zrows)],
                        acc.at[pl.ds(sid * zrows, zrows)])
        pltpu.sync_copy(dst_hbm.at[pl.ds(wid * gpw, gpw)], didx)
        pltpu.sync_copy(ones_hbm, ones_v)
        plsc.subcore_barrier()

        def fire(j, carry):
            pltpu.async_copy(ones_v, acc.at[didx.at[j]], sem, add=True)
            return carry

        lax.fori_loop(0, gpw, fire, 0)

        def drain(j, carry):
            pltpu.make_async_copy(ones_v, acc.at[didx.at[j]], sem).wait()
            return carry

        lax.fori_loop(0, gpw, drain, 0)
        plsc.subcore_barrier()
        pltpu.sync_copy(acc.at[pl.ds(sid * zrows, zrows)],
                        out_hbm.at[pl.ds(cid * n_pad + sid * zrows, zrows)])

    return deg_kernel


K = 128
NBUF = 8


@functools.lru_cache(maxsize=None)
def _make_agg(n_pad, d):
    zrows = n_pad // NS
    e_w = NG0 * CH
    nk = e_w // K
    assert nk % NBUF == 0 and NG0 == NG1

    @functools.partial(
        pl.kernel,
        out_type=jax.ShapeDtypeStruct((NC * n_pad, d), jnp.float32),
        mesh=_sc_mesh(),
        compiler_params=pltpu.CompilerParams(use_tc_tiling_on_sc=False),
        scratch_types=(
            [pltpu.VMEM((e_w,), jnp.int32),
             pltpu.VMEM((nk, K), jnp.int32)]
            + [pltpu.VMEM((K, d), jnp.float32)] * NBUF
            + [pltpu.VMEM_SHARED((n_pad, d), jnp.float32)]
            + [pltpu.SemaphoreType.DMA] * NBUF
        ),
    )
    def agg_kernel(src_hbm, dst_hbm, h_hbm, zero_hbm, out_hbm,
                   sidx, didx, *rest):
        rows = rest[:NBUF]
        acc = rest[NBUF]
        sems = rest[NBUF + 1:]
        cid = lax.axis_index("c")
        sid = lax.axis_index("s")
        wid = cid * NS + sid
        base = wid * e_w
        pltpu.sync_copy(zero_hbm.at[pl.ds(sid * zrows, zrows)],
                        acc.at[pl.ds(sid * zrows, zrows)])
        pltpu.sync_copy(src_hbm.at[pl.ds(base, e_w)], sidx)
        pltpu.sync_copy(dst_hbm.at[pl.ds(base // K, nk)], didx)
        plsc.subcore_barrier()

        def fire(j, buf):
            off = jnp.minimum(j, nk - 1) * K
            pltpu.async_copy(h_hbm.at[sidx.at[pl.ds(off, K)]],
                             rows[buf], sems[buf])

        for b in range(NBUF - 1):
            fire(b, b)

        def body(t, carry):
            for b in range(NBUF):
                j = NBUF * t + b
                fire(j + NBUF - 1, (b + NBUF - 1) % NBUF)
                pltpu.make_async_copy(h_hbm.at[sidx.at[pl.ds(0, K)]],
                                      rows[b], sems[b]).wait()
                pltpu.sync_copy(rows[b], acc.at[didx.at[j]], add=True)
            return carry

        lax.fori_loop(0, nk // NBUF, body, 0)
        for b in range(NBUF - 1):
            pltpu.make_async_copy(h_hbm.at[sidx.at[pl.ds(0, K)]],
                                  rows[b], sems[b]).wait()
        plsc.subcore_barrier()
        pltpu.sync_copy(acc.at[pl.ds(sid * zrows, zrows)],
                        out_hbm.at[pl.ds(cid * n_pad + sid * zrows, zrows)])

    return agg_kernel


def _tc0_body(x_ref, w_ref, xw_ref):
    xw_ref[...] = jnp.dot(x_ref[...], w_ref[...],
                          preferred_element_type=jnp.float32)


def _tc1_body(xw_ref, deg_ref, h_ref, dis_ref):
    deg = deg_ref[0, :, :1] + deg_ref[1, :, :1] + 1.0
    dis = lax.rsqrt(deg)
    dis_ref[...] = dis
    h_ref[...] = dis * xw_ref[...]


def _tc2_body(a_ref, h1s_ref, dis_ref, b1_ref, w2_ref, out_ref):
    dis = dis_ref[...]
    h = dis * (a_ref[0] + a_ref[1] + h1s_ref[...]) + b1_ref[...]
    h = jnp.maximum(h, 0.0)
    out_ref[...] = dis * jnp.dot(h, w2_ref[...],
                                 preferred_element_type=jnp.float32)


def _tc3_body(a_ref, h2s_ref, dis_ref, b2_ref, out_ref):
    dis = dis_ref[...]
    t = dis * (a_ref[0] + a_ref[1] + h2s_ref[...]) + b2_ref[...]
    m = jnp.max(t, axis=1, keepdims=True)
    lse = jnp.log(jnp.sum(jnp.exp(t - m), axis=1, keepdims=True)) + m
    out_ref[...] = t - lse


def kernel(x, edge_index, W1, b1, W2, b2):
    N, d_in = x.shape
    d_h = W1.shape[1]
    d_out = W2.shape[1]
    E = edge_index.shape[1]
    f32 = jnp.float32

    assert E <= NGT * CH
    gpw = NGT // NW
    le = NGT * CH
    n_pad = -(-(N + 1) // 128) * 128
    pad = le - E

    src_p = jnp.concatenate([edge_index[0],
                             jnp.zeros((pad,), edge_index.dtype)])
    dst_f = jnp.concatenate([edge_index[1],
                             jnp.full((pad,), N, edge_index.dtype)])
    dst_p = dst_f.reshape(le // CH, CH)
    dst_a = dst_f.reshape(le // K, K)

    degs = _make_deg(n_pad, gpw)(
        dst_p, jnp.ones((CH, DW), f32), jnp.zeros((n_pad, DW), f32))
    degs3 = degs.reshape(NC, n_pad, DW)

    R = 2000
    grid = (N // R,)
    xw1 = pl.pallas_call(
        _tc0_body,
        grid=grid,
        in_specs=[
            pl.BlockSpec((R, d_in), lambda i: (i, 0)),
            pl.BlockSpec((d_in, d_h), lambda i: (0, 0)),
        ],
        out_specs=pl.BlockSpec((R, d_h), lambda i: (i, 0)),
        out_shape=jax.ShapeDtypeStruct((N, d_h), f32),
    )(x, W1)

    h1s, dis = pl.pallas_call(
        _tc1_body,
        grid=grid,
        in_specs=[
            pl.BlockSpec((R, d_h), lambda i: (i, 0)),
            pl.BlockSpec((NC, R, DW), lambda i: (0, i, 0)),
        ],
        out_specs=[
            pl.BlockSpec((R, d_h), lambda i: (i, 0)),
            pl.BlockSpec((R, 1), lambda i: (i, 0)),
        ],
        out_shape=[
            jax.ShapeDtypeStruct((N, d_h), f32),
            jax.ShapeDtypeStruct((N, 1), f32),
        ],
    )(xw1, degs3)

    agg1 = _make_agg(n_pad, d_h)(
        src_p, dst_a, h1s, jnp.zeros((n_pad, d_h), f32))

    h2s = pl.pallas_call(
        _tc2_body,
        grid=grid,
        in_specs=[
            pl.BlockSpec((NC, R, d_h), lambda i: (0, i, 0)),
            pl.BlockSpec((R, d_h), lambda i: (i, 0)),
            pl.BlockSpec((R, 1), lambda i: (i, 0)),
            pl.BlockSpec((1, d_h), lambda i: (0, 0)),
            pl.BlockSpec((d_h, d_out), lambda i: (0, 0)),
        ],
        out_specs=pl.BlockSpec((R, d_out), lambda i: (i, 0)),
        out_shape=jax.ShapeDtypeStruct((N, d_out), f32),
    )(agg1.reshape(NC, n_pad, d_h), h1s, dis, b1.reshape(1, d_h), W2)

    agg2 = _make_agg(n_pad, d_out)(
        src_p, dst_a, h2s, jnp.zeros((n_pad, d_out), f32))

    out = pl.pallas_call(
        _tc3_body,
        grid=grid,
        in_specs=[
            pl.BlockSpec((NC, R, d_out), lambda i: (0, i, 0)),
            pl.BlockSpec((R, d_out), lambda i: (i, 0)),
            pl.BlockSpec((R, 1), lambda i: (i, 0)),
            pl.BlockSpec((1, d_out), lambda i: (0, 0)),
        ],
        out_specs=pl.BlockSpec((R, d_out), lambda i: (i, 0)),
        out_shape=jax.ShapeDtypeStruct((N, d_out), f32),
    )(agg2.reshape(NC, n_pad, d_out), h2s, dis, b2.reshape(1, d_out))

    return out


# speedup vs baseline: 1.0446x; 1.0446x over previous
import functools

import jax
import jax.numpy as jnp
from jax import lax
from jax.experimental import pallas as pl
from jax.experimental.pallas import tpu as pltpu
from jax.experimental.pallas import tpu_sc as plsc

NC = 2
NS = 16
NW = NC * NS
CH = 640
DW = 16
NG0 = 16
NG1 = 16
NGT = NS * (NG0 + NG1)
NGMAX = max(NG0, NG1)


def _sc_mesh():
    return plsc.VectorSubcoreMesh(core_axis_name="c", subcore_axis_name="s")


@functools.lru_cache(maxsize=None)
def _make_deg(n_pad, gpw):
    zrows = n_pad // NS

    @functools.partial(
        pl.kernel,
        out_type=jax.ShapeDtypeStruct((NC * n_pad, DW), jnp.float32),
        mesh=_sc_mesh(),
        compiler_params=pltpu.CompilerParams(use_tc_tiling_on_sc=False),
        scratch_types=[
            pltpu.VMEM((gpw, CH), jnp.int32),
            pltpu.VMEM((CH, DW), jnp.float32),
            pltpu.VMEM_SHARED((n_pad, DW), jnp.float32),
            pltpu.SemaphoreType.DMA,
        ],
    )
    def deg_kernel(dst_hbm, ones_hbm, zero_hbm, out_hbm, didx, ones_v, acc, sem):
        cid = lax.axis_index("c")
        sid = lax.axis_index("s")
        wid = cid * NS + sid
        pltpu.sync_copy(zero_hbm.at[pl.ds(sid * zrows, zrows)],
                        acc.at[pl.ds(sid * zrows, zrows)])
        pltpu.sync_copy(dst_hbm.at[pl.ds(wid * gpw, gpw)], didx)
        pltpu.sync_copy(ones_hbm, ones_v)
        plsc.subcore_barrier()

        def fire(j, carry):
            pltpu.async_copy(ones_v, acc.at[didx.at[j]], sem, add=True)
            return carry

        lax.fori_loop(0, gpw, fire, 0)

        def drain(j, carry):
            pltpu.make_async_copy(ones_v, acc.at[didx.at[j]], sem).wait()
            return carry

        lax.fori_loop(0, gpw, drain, 0)
        plsc.subcore_barrier()
        pltpu.sync_copy(acc.at[pl.ds(sid * zrows, zrows)],
                        out_hbm.at[pl.ds(cid * n_pad + sid * zrows, zrows)])

    return deg_kernel


K = 128
NBUF = 4


@functools.lru_cache(maxsize=None)
def _make_agg(n_pad, d):
    zrows = n_pad // NS
    e_w = NG0 * CH
    nk = e_w // K
    assert nk % NBUF == 0 and NG0 == NG1

    @functools.partial(
        pl.kernel,
        out_type=jax.ShapeDtypeStruct((NC * n_pad, d), jnp.float32),
        mesh=_sc_mesh(),
        compiler_params=pltpu.CompilerParams(use_tc_tiling_on_sc=False),
        scratch_types=(
            [pltpu.VMEM((e_w,), jnp.int32),
             pltpu.VMEM((nk, K), jnp.int32)]
            + [pltpu.VMEM((K, d), jnp.float32)] * NBUF
            + [pltpu.VMEM_SHARED((n_pad, d), jnp.float32)]
            + [pltpu.SemaphoreType.DMA] * NBUF
        ),
    )
    def agg_kernel(src_hbm, dst_hbm, h_hbm, zero_hbm, out_hbm,
                   sidx, didx, *rest):
        rows = rest[:NBUF]
        acc = rest[NBUF]
        sems = rest[NBUF + 1:]
        cid = lax.axis_index("c")
        sid = lax.axis_index("s")
        wid = cid * NS + sid
        base = wid * e_w
        pltpu.sync_copy(zero_hbm.at[pl.ds(sid * zrows, zrows)],
                        acc.at[pl.ds(sid * zrows, zrows)])
        pltpu.sync_copy(src_hbm.at[pl.ds(base, e_w)], sidx)
        pltpu.sync_copy(dst_hbm.at[pl.ds(base // K, nk)], didx)
        plsc.subcore_barrier()

        def fire(j, buf):
            off = jnp.minimum(j, nk - 1) * K
            pltpu.async_copy(h_hbm.at[sidx.at[pl.ds(off, K)]],
                             rows[buf], sems[buf])

        for b in range(NBUF - 1):
            fire(b, b)

        def body(t, carry):
            for b in range(NBUF):
                j = NBUF * t + b
                fire(j + NBUF - 1, (b + NBUF - 1) % NBUF)
                pltpu.make_async_copy(h_hbm.at[sidx.at[pl.ds(0, K)]],
                                      rows[b], sems[b]).wait()
                pltpu.sync_copy(rows[b], acc.at[didx.at[j]], add=True)
            return carry

        lax.fori_loop(0, nk // NBUF, body, 0)
        for b in range(NBUF - 1):
            pltpu.make_async_copy(h_hbm.at[sidx.at[pl.ds(0, K)]],
                                  rows[b], sems[b]).wait()
        plsc.subcore_barrier()
        pltpu.sync_copy(acc.at[pl.ds(sid * zrows, zrows)],
                        out_hbm.at[pl.ds(cid * n_pad + sid * zrows, zrows)])

    return agg_kernel


def _tc0_body(x_ref, w_ref, xw_ref):
    xw_ref[...] = jnp.dot(x_ref[...], w_ref[...],
                          preferred_element_type=jnp.float32)


def _tc1_body(xw_ref, deg_ref, h_ref, dis_ref):
    deg = deg_ref[0, :, :1] + deg_ref[1, :, :1] + 1.0
    dis = lax.rsqrt(deg)
    dis_ref[...] = dis
    h_ref[...] = dis * xw_ref[...]


def _tc2_body(a_ref, h1s_ref, dis_ref, b1_ref, w2_ref, out_ref):
    dis = dis_ref[...]
    h = dis * (a_ref[0] + a_ref[1] + h1s_ref[...]) + b1_ref[...]
    h = jnp.maximum(h, 0.0)
    out_ref[...] = dis * jnp.dot(h, w2_ref[...],
                                 preferred_element_type=jnp.float32)


def _tc3_body(a_ref, h2s_ref, dis_ref, b2_ref, out_ref):
    dis = dis_ref[...]
    t = dis * (a_ref[0] + a_ref[1] + h2s_ref[...]) + b2_ref[...]
    m = jnp.max(t, axis=1, keepdims=True)
    lse = jnp.log(jnp.sum(jnp.exp(t - m), axis=1, keepdims=True)) + m
    out_ref[...] = t - lse


def kernel(x, edge_index, W1, b1, W2, b2):
    N, d_in = x.shape
    d_h = W1.shape[1]
    d_out = W2.shape[1]
    E = edge_index.shape[1]
    f32 = jnp.float32

    assert E <= NGT * CH
    gpw = NGT // NW
    le = NGT * CH
    n_pad = -(-(N + 1) // 128) * 128
    pad = le - E

    src_p = jnp.concatenate([edge_index[0],
                             jnp.zeros((pad,), edge_index.dtype)])
    dst_f = jnp.concatenate([edge_index[1],
                             jnp.full((pad,), N, edge_index.dtype)])
    dst_p = dst_f.reshape(le // CH, CH)
    dst_a = dst_f.reshape(le // K, K)

    degs = _make_deg(n_pad, gpw)(
        dst_p, jnp.ones((CH, DW), f32), jnp.zeros((n_pad, DW), f32))
    degs3 = degs.reshape(NC, n_pad, DW)

    R = 2000
    grid = (N // R,)
    xw1 = pl.pallas_call(
        _tc0_body,
        grid=grid,
        in_specs=[
            pl.BlockSpec((R, d_in), lambda i: (i, 0)),
            pl.BlockSpec((d_in, d_h), lambda i: (0, 0)),
        ],
        out_specs=pl.BlockSpec((R, d_h), lambda i: (i, 0)),
        out_shape=jax.ShapeDtypeStruct((N, d_h), f32),
    )(x, W1)

    h1s, dis = pl.pallas_call(
        _tc1_body,
        grid=grid,
        in_specs=[
            pl.BlockSpec((R, d_h), lambda i: (i, 0)),
            pl.BlockSpec((NC, R, DW), lambda i: (0, i, 0)),
        ],
        out_specs=[
            pl.BlockSpec((R, d_h), lambda i: (i, 0)),
            pl.BlockSpec((R, 1), lambda i: (i, 0)),
        ],
        out_shape=[
            jax.ShapeDtypeStruct((N, d_h), f32),
            jax.ShapeDtypeStruct((N, 1), f32),
        ],
    )(xw1, degs3)

    agg1 = _make_agg(n_pad, d_h)(
        src_p, dst_a, h1s, jnp.zeros((n_pad, d_h), f32))

    h2s = pl.pallas_call(
        _tc2_body,
        grid=grid,
        in_specs=[
            pl.BlockSpec((NC, R, d_h), lambda i: (0, i, 0)),
            pl.BlockSpec((R, d_h), lambda i: (i, 0)),
            pl.BlockSpec((R, 1), lambda i: (i, 0)),
            pl.BlockSpec((1, d_h), lambda i: (0, 0)),
            pl.BlockSpec((d_h, d_out), lambda i: (0, 0)),
        ],
        out_specs=pl.BlockSpec((R, d_out), lambda i: (i, 0)),
        out_shape=jax.ShapeDtypeStruct((N, d_out), f32),
    )(agg1.reshape(NC, n_pad, d_h), h1s, dis, b1.reshape(1, d_h), W2)

    agg2 = _make_agg(n_pad, d_out)(
        src_p, dst_a, h2s, jnp.zeros((n_pad, d_out), f32))

    out = pl.pallas_call(
        _tc3_body,
        grid=grid,
        in_specs=[
            pl.BlockSpec((NC, R, d_out), lambda i: (0, i, 0)),
            pl.BlockSpec((R, d_out), lambda i: (i, 0)),
            pl.BlockSpec((R, 1), lambda i: (i, 0)),
            pl.BlockSpec((1, d_out), lambda i: (0, 0)),
        ],
        out_specs=pl.BlockSpec((R, d_out), lambda i: (i, 0)),
        out_shape=jax.ShapeDtypeStruct((N, d_out), f32),
    )(agg2.reshape(NC, n_pad, d_out), h2s, dis, b2.reshape(1, d_out))

    return out


# speedup vs baseline: 1.1786x; 1.1283x over previous
import functools

import jax
import jax.numpy as jnp
from jax import lax
from jax.experimental import pallas as pl
from jax.experimental.pallas import tpu as pltpu
from jax.experimental.pallas import tpu_sc as plsc

NC = 2
NS = 16
NW = NC * NS
CH = 640
DW = 16
NG0 = 16
NG1 = 16
NGT = NS * (NG0 + NG1)
NGMAX = max(NG0, NG1)


def _sc_mesh():
    return plsc.VectorSubcoreMesh(core_axis_name="c", subcore_axis_name="s")


@functools.lru_cache(maxsize=None)
def _make_deg(n_pad, gpw):
    zrows = n_pad // NS

    @functools.partial(
        pl.kernel,
        out_type=jax.ShapeDtypeStruct((NC * n_pad, DW), jnp.float32),
        mesh=_sc_mesh(),
        compiler_params=pltpu.CompilerParams(use_tc_tiling_on_sc=False),
        scratch_types=[
            pltpu.VMEM((gpw, CH), jnp.int32),
            pltpu.VMEM((CH, DW), jnp.float32),
            pltpu.VMEM_SHARED((n_pad, DW), jnp.float32),
            pltpu.SemaphoreType.DMA,
        ],
    )
    def deg_kernel(dst_hbm, ones_hbm, zero_hbm, out_hbm, didx, ones_v, acc, sem):
        cid = lax.axis_index("c")
        sid = lax.axis_index("s")
        wid = cid * NS + sid
        pltpu.sync_copy(zero_hbm.at[pl.ds(sid * zrows, zrows)],
                        acc.at[pl.ds(sid * zrows, zrows)])
        pltpu.sync_copy(dst_hbm.at[pl.ds(wid * gpw, gpw)], didx)
        pltpu.sync_copy(ones_hbm, ones_v)
        plsc.subcore_barrier()

        def fire(j, carry):
            pltpu.async_copy(ones_v, acc.at[didx.at[j]], sem, add=True)
            return carry

        lax.fori_loop(0, gpw, fire, 0)

        def drain(j, carry):
            pltpu.make_async_copy(ones_v, acc.at[didx.at[j]], sem).wait()
            return carry

        lax.fori_loop(0, gpw, drain, 0)
        plsc.subcore_barrier()
        pltpu.sync_copy(acc.at[pl.ds(sid * zrows, zrows)],
                        out_hbm.at[pl.ds(cid * n_pad + sid * zrows, zrows)])

    return deg_kernel


K = 128
NBUF = 4


@functools.lru_cache(maxsize=None)
def _make_agg(n_pad, d):
    zrows = n_pad // NS
    e_w = NG0 * CH
    nk = e_w // K
    assert nk % NBUF == 0 and NG0 == NG1

    @functools.partial(
        pl.kernel,
        out_type=jax.ShapeDtypeStruct((NC * n_pad, d), jnp.float32),
        mesh=_sc_mesh(),
        compiler_params=pltpu.CompilerParams(use_tc_tiling_on_sc=False),
        scratch_types=(
            [pltpu.VMEM((e_w,), jnp.int32),
             pltpu.VMEM((nk, K), jnp.int32)]
            + [pltpu.VMEM((K, d), jnp.float32)] * NBUF
            + [pltpu.VMEM_SHARED((n_pad, d), jnp.float32)]
            + [pltpu.SemaphoreType.DMA] * NBUF
        ),
    )
    def agg_kernel(src_hbm, dst_hbm, h_hbm, zero_hbm, out_hbm,
                   sidx, didx, *rest):
        rows = rest[:NBUF]
        acc = rest[NBUF]
        sems = rest[NBUF + 1:]
        cid = lax.axis_index("c")
        sid = lax.axis_index("s")
        wid = cid * NS + sid
        base = wid * e_w
        pltpu.sync_copy(zero_hbm.at[pl.ds(sid * zrows, zrows)],
                        acc.at[pl.ds(sid * zrows, zrows)])
        pltpu.sync_copy(src_hbm.at[pl.ds(base, e_w)], sidx)
        pltpu.sync_copy(dst_hbm.at[pl.ds(base // K, nk)], didx)
        plsc.subcore_barrier()

        def fire(j, buf):
            off = jnp.minimum(j, nk - 1) * K
            pltpu.async_copy(h_hbm.at[sidx.at[pl.ds(off, K)]],
                             rows[buf], sems[buf])

        for b in range(NBUF - 1):
            fire(b, b)

        def body(t, carry):
            for b in range(NBUF):
                j = NBUF * t + b
                fire(j + NBUF - 1, (b + NBUF - 1) % NBUF)
                pltpu.make_async_copy(h_hbm.at[sidx.at[pl.ds(0, K)]],
                                      rows[b], sems[b]).wait()
                pltpu.sync_copy(rows[b], acc.at[didx.at[j]], add=True)
            return carry

        lax.fori_loop(0, nk // NBUF, body, 0)
        for b in range(NBUF - 1):
            pltpu.make_async_copy(h_hbm.at[sidx.at[pl.ds(0, K)]],
                                  rows[b], sems[b]).wait()
        plsc.subcore_barrier()
        pltpu.sync_copy(acc.at[pl.ds(sid * zrows, zrows)],
                        out_hbm.at[pl.ds(cid * n_pad + sid * zrows, zrows)])

    return agg_kernel


def _tc1_body(x_ref, w_ref, deg_ref, h_ref, dis_ref):
    deg = deg_ref[0, :, :1] + deg_ref[1, :, :1] + 1.0
    dis = lax.rsqrt(deg)
    dis_ref[...] = dis
    h_ref[...] = dis * jnp.dot(x_ref[...], w_ref[...],
                               preferred_element_type=jnp.float32)


def _tc2_body(a_ref, h1s_ref, dis_ref, b1_ref, w2_ref, out_ref):
    dis = dis_ref[...]
    h = dis * (a_ref[0] + a_ref[1] + h1s_ref[...]) + b1_ref[...]
    h = jnp.maximum(h, 0.0)
    out_ref[...] = dis * jnp.dot(h, w2_ref[...],
                                 preferred_element_type=jnp.float32)


def _tc3_body(a_ref, h2s_ref, dis_ref, b2_ref, out_ref):
    dis = dis_ref[...]
    t = dis * (a_ref[0] + a_ref[1] + h2s_ref[...]) + b2_ref[...]
    m = jnp.max(t, axis=1, keepdims=True)
    lse = jnp.log(jnp.sum(jnp.exp(t - m), axis=1, keepdims=True)) + m
    out_ref[...] = t - lse


def kernel(x, edge_index, W1, b1, W2, b2):
    N, d_in = x.shape
    d_h = W1.shape[1]
    d_out = W2.shape[1]
    E = edge_index.shape[1]
    f32 = jnp.float32

    assert E <= NGT * CH
    gpw = NGT // NW
    le = NGT * CH
    n_pad = -(-(N + 1) // 128) * 128
    pad = le - E

    src_p = jnp.concatenate([edge_index[0],
                             jnp.zeros((pad,), edge_index.dtype)])
    dst_f = jnp.concatenate([edge_index[1],
                             jnp.full((pad,), N, edge_index.dtype)])
    dst_p = dst_f.reshape(le // CH, CH)
    dst_a = dst_f.reshape(le // K, K)

    degs = _make_deg(n_pad, gpw)(
        dst_p, jnp.ones((CH, DW), f32), jnp.zeros((n_pad, DW), f32))
    degs3 = degs.reshape(NC, n_pad, DW)

    R = 2000
    grid = (N // R,)
    h1s, dis = pl.pallas_call(
        _tc1_body,
        grid=grid,
        in_specs=[
            pl.BlockSpec((R, d_in), lambda i: (i, 0)),
            pl.BlockSpec((d_in, d_h), lambda i: (0, 0)),
            pl.BlockSpec((NC, R, DW), lambda i: (0, i, 0)),
        ],
        out_specs=[
            pl.BlockSpec((R, d_h), lambda i: (i, 0)),
            pl.BlockSpec((R, 1), lambda i: (i, 0)),
        ],
        out_shape=[
            jax.ShapeDtypeStruct((N, d_h), f32),
            jax.ShapeDtypeStruct((N, 1), f32),
        ],
    )(x, W1, degs3)

    agg1 = _make_agg(n_pad, d_h)(
        src_p, dst_a, h1s, jnp.zeros((n_pad, d_h), f32))

    h2s = pl.pallas_call(
        _tc2_body,
        grid=grid,
        in_specs=[
            pl.BlockSpec((NC, R, d_h), lambda i: (0, i, 0)),
            pl.BlockSpec((R, d_h), lambda i: (i, 0)),
            pl.BlockSpec((R, 1), lambda i: (i, 0)),
            pl.BlockSpec((1, d_h), lambda i: (0, 0)),
            pl.BlockSpec((d_h, d_out), lambda i: (0, 0)),
        ],
        out_specs=pl.BlockSpec((R, d_out), lambda i: (i, 0)),
        out_shape=jax.ShapeDtypeStruct((N, d_out), f32),
    )(agg1.reshape(NC, n_pad, d_h), h1s, dis, b1.reshape(1, d_h), W2)

    agg2 = _make_agg(n_pad, d_out)(
        src_p, dst_a, h2s, jnp.zeros((n_pad, d_out), f32))

    out = pl.pallas_call(
        _tc3_body,
        grid=grid,
        in_specs=[
            pl.BlockSpec((NC, R, d_out), lambda i: (0, i, 0)),
            pl.BlockSpec((R, d_out), lambda i: (i, 0)),
            pl.BlockSpec((R, 1), lambda i: (i, 0)),
            pl.BlockSpec((1, d_out), lambda i: (0, 0)),
        ],
        out_specs=pl.BlockSpec((R, d_out), lambda i: (i, 0)),
        out_shape=jax.ShapeDtypeStruct((N, d_out), f32),
    )(agg2.reshape(NC, n_pad, d_out), h2s, dis, b2.reshape(1, d_out))

    return out
